# baseline (speedup 1.0000x reference)
import jax
import jax.numpy as jnp
from jax import lax
from jax.experimental import pallas as pl
from jax.experimental.pallas import tpu as pltpu
from jax.experimental.pallas import tpu_sc as plsc

B = 16
L = 2048
S = 512
V = 50265
D = 1024

NC = 2
NS = 16
LANES = 16

DCW = 128
NDC = D // DCW
TCHUNK = 128
NCHUNK = L // TCHUNK
R = 2 * B
NV = L // LANES
EPAD = LANES


ZROWS = 32


def _body(tok_hbm, seg_hbm, table_hbm, out_hbm, cnt_hbm,
          tok_v, seg_v, sidx_v, idx_v, gbuf_a, gbuf_b, obuf, zbuf,
          seg_f, ends_v, cntbuf, inv_v,
          acc, gs_a, gs_b, ss_a, ss_b, zsem):
    s = lax.axis_index("s")
    r = s
    base = s * S
    iota = lax.iota(jnp.int32, LANES)

    pltpu.sync_copy(tok_hbm.at[r], tok_v)
    pltpu.sync_copy(seg_hbm.at[r], seg_v)

    def mk_sidx(j, _):
        def mk_k(k, _):
            sl = pl.ds(k * LANES, LANES)
            sidx_v[j, sl] = seg_v[j, sl] + base
            return 0
        return lax.fori_loop(0, TCHUNK // LANES, mk_k, 0)
    lax.fori_loop(0, NCHUNK, mk_sidx, 0)

    def zero_ends(i, _):
        ends_v[pl.ds(i * LANES, LANES)] = jnp.zeros((LANES,), jnp.int32)
        return 0
    lax.fori_loop(0, (EPAD + S) // LANES, zero_ends, 0)

    def mk_flat(j, _):
        def fk(k, _):
            seg_f[pl.ds(j * TCHUNK + k * LANES, LANES)] = \
                seg_v[j, pl.ds(k * LANES, LANES)]
            return 0
        return lax.fori_loop(0, TCHUNK // LANES, fk, 0)
    lax.fori_loop(0, NCHUNK, mk_flat, 0)

    def bounds(t, _):
        cur = seg_f[pl.ds(t * LANES, LANES)]
        pos = t * LANES + iota
        pnx = jnp.minimum(pos + 1, L - 1)
        nxt = plsc.load_gather(seg_f, [pnx])
        mask = jnp.logical_or(cur != nxt, pos == L - 1)
        plsc.store_scatter(ends_v, [cur + EPAD], pos + 1, mask=mask)
        return 0
    lax.fori_loop(0, NV, bounds, 0)

    def fill(i, carry):
        sl = pl.ds(EPAD + i * LANES, LANES)
        filled = jnp.maximum(plsc.cummax(ends_v[sl]),
                             jnp.full((LANES,), carry, jnp.int32))
        ends_v[sl] = filled
        prev = plsc.load_gather(ends_v, [EPAD - 1 + i * LANES + iota])
        cnt_i = filled - prev
        cntbuf[0, pl.ds(i * LANES, LANES)] = cnt_i
        inv_v[pl.ds(i * LANES, LANES)] = 1.0 / jnp.maximum(
            cnt_i.astype(jnp.float32), 1.0)
        return jnp.max(filled)
    lax.fori_loop(0, S // LANES, fill, jnp.int32(0))

    pltpu.sync_copy(cntbuf, cnt_hbm.at[r])

    def zero_zb(i, _):
        def zk(k, _):
            zbuf[i, pl.ds(k * LANES, LANES)] = jnp.zeros((LANES,), jnp.float32)
            return 0
        return lax.fori_loop(0, DCW // LANES, zk, 0)
    lax.fori_loop(0, ZROWS, zero_zb, 0)

    def dc_body(dc, _):
        for m in range(S // ZROWS):
            pltpu.async_copy(zbuf, acc.at[pl.ds(base + m * ZROWS, ZROWS)], zsem)

        def mk_idx(j, _):
            def mk_k(k, _):
                sl = pl.ds(k * LANES, LANES)
                idx_v[j, sl] = tok_v[j, sl] * NDC + dc
                return 0
            return lax.fori_loop(0, TCHUNK // LANES, mk_k, 0)
        lax.fori_loop(0, NCHUNK, mk_idx, 0)

        def gadd(i, _):
            j0 = 2 * i
            j1 = 2 * i + 1

            @pl.when(i >= 1)
            def _():
                pltpu.make_async_copy(
                    gbuf_a, acc.at[pl.ds(base, TCHUNK)], ss_a).wait()
            gd_a = pltpu.async_copy(table_hbm.at[idx_v.at[j0]], gbuf_a, gs_a)

            @pl.when(i >= 1)
            def _():
                pltpu.make_async_copy(
                    gbuf_b, acc.at[pl.ds(base, TCHUNK)], ss_b).wait()
            gd_b = pltpu.async_copy(table_hbm.at[idx_v.at[j1]], gbuf_b, gs_b)

            @pl.when(i == 0)
            def _():
                for m in range(S // ZROWS):
                    pltpu.make_async_copy(
                        zbuf, acc.at[pl.ds(base, ZROWS)], zsem).wait()

            gd_a.wait()
            pltpu.async_copy(gbuf_a, acc.at[sidx_v.at[j0]], ss_a, add=True)
            gd_b.wait()
            pltpu.async_copy(gbuf_b, acc.at[sidx_v.at[j1]], ss_b, add=True)
            return 0
        lax.fori_loop(0, NCHUNK // 2, gadd, 0)

        pltpu.make_async_copy(gbuf_a, acc.at[pl.ds(base, TCHUNK)], ss_a).wait()
        pltpu.make_async_copy(gbuf_b, acc.at[pl.ds(base, TCHUNK)], ss_b).wait()

        def scale(t, _):
            pltpu.sync_copy(acc.at[pl.ds(base + t * ZROWS, ZROWS)], obuf)
            for g in range(ZROWS // LANES):
                invv = inv_v[pl.ds(t * ZROWS + g * LANES, LANES)]
                for i2 in range(LANES):
                    splat = jnp.full((LANES,), invv[i2], jnp.float32)
                    for k in range(DCW // LANES):
                        sl = pl.ds(k * LANES, LANES)
                        obuf[g * LANES + i2, sl] = obuf[g * LANES + i2, sl] * splat
            pltpu.sync_copy(
                obuf,
                out_hbm.at[r, pl.ds(t * ZROWS, ZROWS), pl.ds(dc * DCW, DCW)])
            return 0
        lax.fori_loop(0, S // ZROWS, scale, 0)
        return 0
    lax.fori_loop(0, NDC, dc_body, 0)


def _pooled(tok2, seg2, table):
    mesh = plsc.VectorSubcoreMesh(
        core_axis_name="c", subcore_axis_name="s", num_cores=1, num_subcores=NS)
    kern = pl.kernel(
        _body,
        out_type=(
            jax.ShapeDtypeStruct((B, S, D), jnp.float32),
            jax.ShapeDtypeStruct((B, 1, S), jnp.int32),
        ),
        mesh=mesh,
        compiler_params=pltpu.CompilerParams(needs_layout_passes=False),
        scratch_types=[
            pltpu.VMEM((NCHUNK, TCHUNK), jnp.int32),
            pltpu.VMEM((NCHUNK, TCHUNK), jnp.int32),
            pltpu.VMEM((NCHUNK, TCHUNK), jnp.int32),
            pltpu.VMEM((NCHUNK, TCHUNK), jnp.int32),
            pltpu.VMEM((TCHUNK, DCW), jnp.float32),
            pltpu.VMEM((TCHUNK, DCW), jnp.float32),
            pltpu.VMEM((ZROWS, DCW), jnp.float32),
            pltpu.VMEM((ZROWS, DCW), jnp.float32),
            pltpu.VMEM((L,), jnp.int32),
            pltpu.VMEM((EPAD + S,), jnp.int32),
            pltpu.VMEM((1, S), jnp.int32),
            pltpu.VMEM((S,), jnp.float32),
            pltpu.VMEM_SHARED((NS * S, DCW), jnp.float32),
            pltpu.SemaphoreType.DMA,
            pltpu.SemaphoreType.DMA,
            pltpu.SemaphoreType.DMA,
            pltpu.SemaphoreType.DMA,
            pltpu.SemaphoreType.DMA,
        ],
    )
    return kern(tok2, seg2, table)


def kernel(text_token_ids, text_seg_ids, amr_token_ids, amr_seg_ids, table):
    table8 = table.reshape(V * NDC, DCW)
    tt = text_token_ids.astype(jnp.int32).reshape(B, NCHUNK, TCHUNK)
    ts = text_seg_ids.astype(jnp.int32).reshape(B, NCHUNK, TCHUNK)
    at = amr_token_ids.astype(jnp.int32).reshape(B, NCHUNK, TCHUNK)
    asg = amr_seg_ids.astype(jnp.int32).reshape(B, NCHUNK, TCHUNK)
    tf, tc = _pooled(tt, ts, table8)
    af, ac = _pooled(at, asg, table8)
    return tf, tc[:, 0, :] == 0, af, ac[:, 0, :] == 0


# speedup vs baseline: 1.4081x; 1.4081x over previous
import jax
import jax.numpy as jnp
from jax import lax
from jax.experimental import pallas as pl
from jax.experimental.pallas import tpu as pltpu
from jax.experimental.pallas import tpu_sc as plsc

B = 16
L = 2048
S = 512
V = 50265
D = 1024

NC = 2
NS = 16
LANES = 16

DCW = 128
NDC = D // DCW
TCHUNK = 128
NCHUNK = L // TCHUNK
R = 2 * B
NV = L // LANES
EPAD = LANES


ZROWS = 32


def _body(tok_hbm, seg_hbm, table_hbm, out_hbm, cnt_hbm,
          tok_v, seg_v, sidx_v, idx_v, gbuf_a, gbuf_b, zbuf,
          seg_f, ends_v, cntbuf, inv_v,
          acc, gs_a, gs_b, ss_a, ss_b, zsem):
    c = lax.axis_index("c")
    s = lax.axis_index("s")
    r = c * NS + s
    base = s * S
    iota = lax.iota(jnp.int32, LANES)

    pltpu.sync_copy(tok_hbm.at[r], tok_v)
    pltpu.sync_copy(seg_hbm.at[r], seg_v)

    def mk_sidx(j, _):
        def mk_k(k, _):
            sl = pl.ds(k * LANES, LANES)
            sidx_v[j, sl] = seg_v[j, sl] + base
            return 0
        return lax.fori_loop(0, TCHUNK // LANES, mk_k, 0)
    lax.fori_loop(0, NCHUNK, mk_sidx, 0)

    def zero_ends(i, _):
        ends_v[pl.ds(i * LANES, LANES)] = jnp.zeros((LANES,), jnp.int32)
        return 0
    lax.fori_loop(0, (EPAD + S) // LANES, zero_ends, 0)

    def mk_flat(j, _):
        def fk(k, _):
            seg_f[pl.ds(j * TCHUNK + k * LANES, LANES)] = \
                seg_v[j, pl.ds(k * LANES, LANES)]
            return 0
        return lax.fori_loop(0, TCHUNK // LANES, fk, 0)
    lax.fori_loop(0, NCHUNK, mk_flat, 0)

    def bounds(t, _):
        cur = seg_f[pl.ds(t * LANES, LANES)]
        pos = t * LANES + iota
        pnx = jnp.minimum(pos + 1, L - 1)
        nxt = plsc.load_gather(seg_f, [pnx])
        mask = jnp.logical_or(cur != nxt, pos == L - 1)
        plsc.store_scatter(ends_v, [cur + EPAD], pos + 1, mask=mask)
        return 0
    lax.fori_loop(0, NV, bounds, 0)

    def fill(i, carry):
        sl = pl.ds(EPAD + i * LANES, LANES)
        filled = jnp.maximum(plsc.cummax(ends_v[sl]),
                             jnp.full((LANES,), carry, jnp.int32))
        ends_v[sl] = filled
        prev = plsc.load_gather(ends_v, [EPAD - 1 + i * LANES + iota])
        cnt_i = filled - prev
        cntbuf[0, pl.ds(i * LANES, LANES)] = cnt_i
        inv_v[pl.ds(i * LANES, LANES)] = 1.0 / jnp.maximum(
            cnt_i.astype(jnp.float32), 1.0)
        return jnp.max(filled)
    lax.fori_loop(0, S // LANES, fill, jnp.int32(0))

    pltpu.sync_copy(cntbuf, cnt_hbm.at[r])

    def zero_zb(i, _):
        def zk(k, _):
            zbuf[i, pl.ds(k * LANES, LANES)] = jnp.zeros((LANES,), jnp.float32)
            return 0
        return lax.fori_loop(0, DCW // LANES, zk, 0)
    lax.fori_loop(0, TCHUNK, zero_zb, 0)

    for m in range(S // TCHUNK):
        pltpu.async_copy(zbuf, acc.at[pl.ds(base + m * TCHUNK, TCHUNK)], zsem)

    def dc_body(dc, _):
        def mk_idx(j, _):
            def mk_k(k, _):
                sl = pl.ds(k * LANES, LANES)
                idx_v[j, sl] = tok_v[j, sl] * NDC + dc
                return 0
            return lax.fori_loop(0, TCHUNK // LANES, mk_k, 0)
        lax.fori_loop(0, NCHUNK, mk_idx, 0)

        def gadd(i, _):
            j0 = 2 * i
            j1 = 2 * i + 1

            @pl.when(i >= 1)
            def _():
                pltpu.make_async_copy(
                    gbuf_a, acc.at[pl.ds(base, TCHUNK)], ss_a).wait()
            gd_a = pltpu.async_copy(table_hbm.at[idx_v.at[j0]], gbuf_a, gs_a)

            @pl.when(i >= 1)
            def _():
                pltpu.make_async_copy(
                    gbuf_b, acc.at[pl.ds(base, TCHUNK)], ss_b).wait()
            gd_b = pltpu.async_copy(table_hbm.at[idx_v.at[j1]], gbuf_b, gs_b)

            @pl.when(i == 0)
            def _():
                for m in range(S // TCHUNK):
                    pltpu.make_async_copy(
                        zbuf, acc.at[pl.ds(base, TCHUNK)], zsem).wait()

            gd_a.wait()
            pltpu.async_copy(gbuf_a, acc.at[sidx_v.at[j0]], ss_a, add=True)
            gd_b.wait()
            pltpu.async_copy(gbuf_b, acc.at[sidx_v.at[j1]], ss_b, add=True)
            return 0
        lax.fori_loop(0, NCHUNK // 2, gadd, 0)

        pltpu.make_async_copy(gbuf_a, acc.at[pl.ds(base, TCHUNK)], ss_a).wait()
        pltpu.make_async_copy(gbuf_b, acc.at[pl.ds(base, TCHUNK)], ss_b).wait()

        NT = S // TCHUNK
        bufs = (gbuf_a, gbuf_b)
        isems = (gs_a, gs_b)
        osems = (ss_a, ss_b)
        dins = {}
        dins[0] = pltpu.async_copy(
            acc.at[pl.ds(base + 0 * TCHUNK, TCHUNK)], gbuf_a, gs_a)
        dins[1] = pltpu.async_copy(
            acc.at[pl.ds(base + 1 * TCHUNK, TCHUNK)], gbuf_b, gs_b)
        for tt in range(NT):
            buf = bufs[tt % 2]
            dins[tt].wait()

            def srow(g, _):
                invv = inv_v[pl.ds(tt * TCHUNK + g * LANES, LANES)]
                for i2 in range(LANES):
                    splat = jnp.full((LANES,), invv[i2], jnp.float32)
                    for k in range(DCW // LANES):
                        sl = pl.ds(k * LANES, LANES)
                        buf[g * LANES + i2, sl] = buf[g * LANES + i2, sl] * splat
                return 0
            lax.fori_loop(0, TCHUNK // LANES, srow, 0)

            pltpu.async_copy(
                zbuf, acc.at[pl.ds(base + tt * TCHUNK, TCHUNK)], zsem)
            dout = pltpu.async_copy(
                buf,
                out_hbm.at[r, pl.ds(tt * TCHUNK, TCHUNK), pl.ds(dc * DCW, DCW)],
                osems[tt % 2])
            if tt + 2 < NT:
                dout.wait()
                dins[tt + 2] = pltpu.async_copy(
                    acc.at[pl.ds(base + (tt + 2) * TCHUNK, TCHUNK)],
                    buf, isems[tt % 2])
            else:
                dout.wait()
        return 0
    lax.fori_loop(0, NDC, dc_body, 0)

    for m in range(S // TCHUNK):
        pltpu.make_async_copy(zbuf, acc.at[pl.ds(base, TCHUNK)], zsem).wait()


def _pooled(tok2, seg2, table):
    mesh = plsc.VectorSubcoreMesh(
        core_axis_name="c", subcore_axis_name="s", num_cores=NC, num_subcores=NS)
    kern = pl.kernel(
        _body,
        out_type=(
            jax.ShapeDtypeStruct((R, S, D), jnp.float32),
            jax.ShapeDtypeStruct((R, 1, S), jnp.int32),
        ),
        mesh=mesh,
        compiler_params=pltpu.CompilerParams(needs_layout_passes=False),
        scratch_types=[
            pltpu.VMEM((NCHUNK, TCHUNK), jnp.int32),
            pltpu.VMEM((NCHUNK, TCHUNK), jnp.int32),
            pltpu.VMEM((NCHUNK, TCHUNK), jnp.int32),
            pltpu.VMEM((NCHUNK, TCHUNK), jnp.int32),
            pltpu.VMEM((TCHUNK, DCW), jnp.float32),
            pltpu.VMEM((TCHUNK, DCW), jnp.float32),
            pltpu.VMEM((TCHUNK, DCW), jnp.float32),
            pltpu.VMEM((L,), jnp.int32),
            pltpu.VMEM((EPAD + S,), jnp.int32),
            pltpu.VMEM((1, S), jnp.int32),
            pltpu.VMEM((S,), jnp.float32),
            pltpu.VMEM_SHARED((NS * S, DCW), jnp.float32),
            pltpu.SemaphoreType.DMA,
            pltpu.SemaphoreType.DMA,
            pltpu.SemaphoreType.DMA,
            pltpu.SemaphoreType.DMA,
            pltpu.SemaphoreType.DMA,
        ],
    )
    return kern(tok2, seg2, table)


def kernel(text_token_ids, text_seg_ids, amr_token_ids, amr_seg_ids, table):
    table8 = table.reshape(V * NDC, DCW)
    tok2 = jnp.concatenate(
        [text_token_ids.astype(jnp.int32), amr_token_ids.astype(jnp.int32)], axis=0
    ).reshape(R, NCHUNK, TCHUNK)
    seg2 = jnp.concatenate(
        [text_seg_ids.astype(jnp.int32), amr_seg_ids.astype(jnp.int32)], axis=0
    ).reshape(R, NCHUNK, TCHUNK)
    feats, cnts = _pooled(tok2, seg2, table8)
    pad = cnts[:, 0, :] == 0
    return feats[:B], pad[:B], feats[B:], pad[B:]


# speedup vs baseline: 1.4254x; 1.0122x over previous
import jax
import jax.numpy as jnp
from jax import lax
from jax.experimental import pallas as pl
from jax.experimental.pallas import tpu as pltpu
from jax.experimental.pallas import tpu_sc as plsc

B = 16
L = 2048
S = 512
V = 50265
D = 1024

NC = 2
NS = 16
LANES = 16

DCW = 128
NDC = D // DCW
TCHUNK = 128
NCHUNK = L // TCHUNK
R = 2 * B
NV = L // LANES
EPAD = LANES


ZROWS = 32


def _body(tok_hbm, seg_hbm, table_hbm, out_hbm, cnt_hbm,
          tok_v, seg_v, sidx_v, idx_v, gbuf_a, gbuf_b, zbuf,
          seg_f, ends_v, cntbuf, inv_v,
          acc, gs_a, gs_b, ss_a, ss_b, zsem, is_s, os_s):
    c = lax.axis_index("c")
    s = lax.axis_index("s")
    r = c * NS + s
    base = s * S
    iota = lax.iota(jnp.int32, LANES)

    pltpu.sync_copy(tok_hbm.at[r], tok_v)
    pltpu.sync_copy(seg_hbm.at[r], seg_v)

    def mk_sidx(j, _):
        def mk_k(k, _):
            sl = pl.ds(k * LANES, LANES)
            sidx_v[j, sl] = seg_v[j, sl] + base
            return 0
        return lax.fori_loop(0, TCHUNK // LANES, mk_k, 0)
    lax.fori_loop(0, NCHUNK, mk_sidx, 0)

    def zero_ends(i, _):
        ends_v[pl.ds(i * LANES, LANES)] = jnp.zeros((LANES,), jnp.int32)
        return 0
    lax.fori_loop(0, (EPAD + S) // LANES, zero_ends, 0)

    def mk_flat(j, _):
        def fk(k, _):
            seg_f[pl.ds(j * TCHUNK + k * LANES, LANES)] = \
                seg_v[j, pl.ds(k * LANES, LANES)]
            return 0
        return lax.fori_loop(0, TCHUNK // LANES, fk, 0)
    lax.fori_loop(0, NCHUNK, mk_flat, 0)

    def bounds(t, _):
        cur = seg_f[pl.ds(t * LANES, LANES)]
        pos = t * LANES + iota
        pnx = jnp.minimum(pos + 1, L - 1)
        nxt = plsc.load_gather(seg_f, [pnx])
        mask = jnp.logical_or(cur != nxt, pos == L - 1)
        plsc.store_scatter(ends_v, [cur + EPAD], pos + 1, mask=mask)
        return 0
    lax.fori_loop(0, NV, bounds, 0)

    def fill(i, carry):
        sl = pl.ds(EPAD + i * LANES, LANES)
        filled = jnp.maximum(plsc.cummax(ends_v[sl]),
                             jnp.full((LANES,), carry, jnp.int32))
        ends_v[sl] = filled
        prev = plsc.load_gather(ends_v, [EPAD - 1 + i * LANES + iota])
        cnt_i = filled - prev
        cntbuf[0, pl.ds(i * LANES, LANES)] = cnt_i
        inv_v[pl.ds(i * LANES, LANES)] = 1.0 / jnp.maximum(
            cnt_i.astype(jnp.float32), 1.0)
        return jnp.max(filled)
    lax.fori_loop(0, S // LANES, fill, jnp.int32(0))

    pltpu.sync_copy(cntbuf, cnt_hbm.at[r])

    def zero_zb(i, _):
        def zk(k, _):
            zbuf[i, pl.ds(k * LANES, LANES)] = jnp.zeros((LANES,), jnp.float32)
            return 0
        return lax.fori_loop(0, DCW // LANES, zk, 0)
    lax.fori_loop(0, TCHUNK, zero_zb, 0)

    for m in range(S // 64):
        pltpu.async_copy(zbuf.at[pl.ds(0, 64)],
                         acc.at[pl.ds(base + m * 64, 64)], zsem)

    def dc_body(dc, _):
        def mk_idx(j, _):
            def mk_k(k, _):
                sl = pl.ds(k * LANES, LANES)
                idx_v[j, sl] = tok_v[j, sl] * NDC + dc
                return 0
            return lax.fori_loop(0, TCHUNK // LANES, mk_k, 0)
        lax.fori_loop(0, NCHUNK, mk_idx, 0)

        def gadd(i, _):
            j0 = 2 * i
            j1 = 2 * i + 1

            @pl.when(i >= 1)
            def _():
                pltpu.make_async_copy(
                    gbuf_a, acc.at[pl.ds(base, TCHUNK)], ss_a).wait()
            gd_a = pltpu.async_copy(table_hbm.at[idx_v.at[j0]], gbuf_a, gs_a)

            @pl.when(i >= 1)
            def _():
                pltpu.make_async_copy(
                    gbuf_b, acc.at[pl.ds(base, TCHUNK)], ss_b).wait()
            gd_b = pltpu.async_copy(table_hbm.at[idx_v.at[j1]], gbuf_b, gs_b)

            @pl.when(i == 0)
            def _():
                for m in range(S // 64):
                    pltpu.make_async_copy(
                        zbuf.at[pl.ds(0, 64)], acc.at[pl.ds(base, 64)],
                        zsem).wait()

            gd_a.wait()
            pltpu.async_copy(gbuf_a, acc.at[sidx_v.at[j0]], ss_a, add=True)
            gd_b.wait()
            pltpu.async_copy(gbuf_b, acc.at[sidx_v.at[j1]], ss_b, add=True)
            return 0
        lax.fori_loop(0, NCHUNK // 2, gadd, 0)

        pltpu.make_async_copy(gbuf_a, acc.at[pl.ds(base, TCHUNK)], ss_a).wait()
        pltpu.make_async_copy(gbuf_b, acc.at[pl.ds(base, TCHUNK)], ss_b).wait()

        SROWS = 64
        NT = S // SROWS
        bufs = (gbuf_a.at[pl.ds(0, SROWS)], gbuf_a.at[pl.ds(SROWS, SROWS)],
                gbuf_b.at[pl.ds(0, SROWS)], gbuf_b.at[pl.ds(SROWS, SROWS)])

        def din(tt):
            return pltpu.async_copy(
                acc.at[pl.ds(base + tt * SROWS, SROWS)], bufs[tt % 4],
                is_s[tt % 4])

        def dout(tt):
            return pltpu.async_copy(
                bufs[tt % 4],
                out_hbm.at[r, pl.ds(tt * SROWS, SROWS), pl.ds(dc * DCW, DCW)],
                os_s[tt % 4])

        dins = {0: din(0), 1: din(1)}
        douts = {}
        for tt in range(NT):
            q = tt % 4
            if tt + 2 < NT:
                if tt + 2 >= 4:
                    douts[tt - 2].wait()
                dins[tt + 2] = din(tt + 2)
            dins[tt].wait()
            buf = bufs[q]

            def srow(g, _):
                invv = inv_v[pl.ds(tt * SROWS + g * LANES, LANES)]
                for i2 in range(LANES):
                    splat = jnp.full((LANES,), invv[i2], jnp.float32)
                    for k in range(DCW // LANES):
                        sl = pl.ds(k * LANES, LANES)
                        buf[g * LANES + i2, sl] = buf[g * LANES + i2, sl] * splat
                return 0
            lax.fori_loop(0, SROWS // LANES, srow, 0)

            pltpu.async_copy(
                zbuf.at[pl.ds(0, SROWS)],
                acc.at[pl.ds(base + tt * SROWS, SROWS)], zsem)
            douts[tt] = dout(tt)
        for tt in range(NT - 4, NT):
            douts[tt].wait()
        return 0
    lax.fori_loop(0, NDC, dc_body, 0)

    for m in range(S // 64):
        pltpu.make_async_copy(zbuf.at[pl.ds(0, 64)], acc.at[pl.ds(base, 64)],
                              zsem).wait()


def _pooled(tok2, seg2, table):
    mesh = plsc.VectorSubcoreMesh(
        core_axis_name="c", subcore_axis_name="s", num_cores=NC, num_subcores=NS)
    kern = pl.kernel(
        _body,
        out_type=(
            jax.ShapeDtypeStruct((R, S, D), jnp.float32),
            jax.ShapeDtypeStruct((R, 1, S), jnp.int32),
        ),
        mesh=mesh,
        compiler_params=pltpu.CompilerParams(needs_layout_passes=False),
        scratch_types=[
            pltpu.VMEM((NCHUNK, TCHUNK), jnp.int32),
            pltpu.VMEM((NCHUNK, TCHUNK), jnp.int32),
            pltpu.VMEM((NCHUNK, TCHUNK), jnp.int32),
            pltpu.VMEM((NCHUNK, TCHUNK), jnp.int32),
            pltpu.VMEM((TCHUNK, DCW), jnp.float32),
            pltpu.VMEM((TCHUNK, DCW), jnp.float32),
            pltpu.VMEM((TCHUNK, DCW), jnp.float32),
            pltpu.VMEM((L,), jnp.int32),
            pltpu.VMEM((EPAD + S,), jnp.int32),
            pltpu.VMEM((1, S), jnp.int32),
            pltpu.VMEM((S,), jnp.float32),
            pltpu.VMEM_SHARED((NS * S, DCW), jnp.float32),
            pltpu.SemaphoreType.DMA,
            pltpu.SemaphoreType.DMA,
            pltpu.SemaphoreType.DMA,
            pltpu.SemaphoreType.DMA,
            pltpu.SemaphoreType.DMA,
            [pltpu.SemaphoreType.DMA] * 4,
            [pltpu.SemaphoreType.DMA] * 4,
        ],
    )
    return kern(tok2, seg2, table)


def kernel(text_token_ids, text_seg_ids, amr_token_ids, amr_seg_ids, table):
    table8 = table.reshape(V * NDC, DCW)
    tok2 = jnp.concatenate(
        [text_token_ids.astype(jnp.int32), amr_token_ids.astype(jnp.int32)], axis=0
    ).reshape(R, NCHUNK, TCHUNK)
    seg2 = jnp.concatenate(
        [text_seg_ids.astype(jnp.int32), amr_seg_ids.astype(jnp.int32)], axis=0
    ).reshape(R, NCHUNK, TCHUNK)
    feats, cnts = _pooled(tok2, seg2, table8)
    pad = cnts[:, 0, :] == 0
    return feats[:B], pad[:B], feats[B:], pad[B:]


# speedup vs baseline: 1.5583x; 1.0933x over previous
import jax
import jax.numpy as jnp
from jax import lax
from jax.experimental import pallas as pl
from jax.experimental.pallas import tpu as pltpu
from jax.experimental.pallas import tpu_sc as plsc

B = 16
L = 2048
S = 512
V = 50265
D = 1024

NC = 2
NS = 16
LANES = 16

DCW = 128
NDC = D // DCW
TCHUNK = 64
NCHUNK = L // TCHUNK
NBUF = 4
SROWS = 64
R = 2 * B
NV = L // LANES
EPAD = LANES


def _body(tok_hbm, seg_hbm, table_hbm, out_hbm, cnt_hbm,
          tok_v, seg_v, sidx_v, idx_v, gb0, gb1, gb2, gb3, zbuf,
          seg_f, ends_v, cntbuf, inv_v,
          acc, gsems, ssems, osems, zsem):
    c = lax.axis_index("c")
    s = lax.axis_index("s")
    r = c * NS + s
    base = s * S
    iota = lax.iota(jnp.int32, LANES)
    gbufs = (gb0, gb1, gb2, gb3)

    pltpu.sync_copy(tok_hbm.at[r], tok_v)
    pltpu.sync_copy(seg_hbm.at[r], seg_v)

    def mk_sidx(j, _):
        def mk_k(k, _):
            sl = pl.ds(k * LANES, LANES)
            sidx_v[j, sl] = seg_v[j, sl] + base
            return 0
        return lax.fori_loop(0, TCHUNK // LANES, mk_k, 0)
    lax.fori_loop(0, NCHUNK, mk_sidx, 0)

    def zero_ends(i, _):
        ends_v[pl.ds(i * LANES, LANES)] = jnp.zeros((LANES,), jnp.int32)
        return 0
    lax.fori_loop(0, (EPAD + S) // LANES, zero_ends, 0)

    def mk_flat(j, _):
        def fk(k, _):
            seg_f[pl.ds(j * TCHUNK + k * LANES, LANES)] = \
                seg_v[j, pl.ds(k * LANES, LANES)]
            return 0
        return lax.fori_loop(0, TCHUNK // LANES, fk, 0)
    lax.fori_loop(0, NCHUNK, mk_flat, 0)

    def bounds(t, _):
        cur = seg_f[pl.ds(t * LANES, LANES)]
        pos = t * LANES + iota
        pnx = jnp.minimum(pos + 1, L - 1)
        nxt = plsc.load_gather(seg_f, [pnx])
        mask = jnp.logical_or(cur != nxt, pos == L - 1)
        plsc.store_scatter(ends_v, [cur + EPAD], pos + 1, mask=mask)
        return 0
    lax.fori_loop(0, NV, bounds, 0)

    def fill(i, carry):
        sl = pl.ds(EPAD + i * LANES, LANES)
        filled = jnp.maximum(plsc.cummax(ends_v[sl]),
                             jnp.full((LANES,), carry, jnp.int32))
        ends_v[sl] = filled
        prev = plsc.load_gather(ends_v, [EPAD - 1 + i * LANES + iota])
        cnt_i = filled - prev
        cntbuf[0, pl.ds(i * LANES, LANES)] = cnt_i
        inv_v[pl.ds(i * LANES, LANES)] = 1.0 / jnp.maximum(
            cnt_i.astype(jnp.float32), 1.0)
        return jnp.max(filled)
    lax.fori_loop(0, S // LANES, fill, jnp.int32(0))

    pltpu.sync_copy(cntbuf, cnt_hbm.at[r])

    def zero_zb(i, _):
        def zk(k, _):
            zbuf[i, pl.ds(k * LANES, LANES)] = jnp.zeros((LANES,), jnp.float32)
            return 0
        return lax.fori_loop(0, DCW // LANES, zk, 0)
    lax.fori_loop(0, SROWS, zero_zb, 0)

    for m in range(S // SROWS):
        pltpu.async_copy(zbuf, acc.at[pl.ds(base + m * SROWS, SROWS)], zsem)

    def dc_body(dc, _):
        def mk_idx(j, _):
            def mk_k(k, _):
                sl = pl.ds(k * LANES, LANES)
                idx_v[j, sl] = tok_v[j, sl] * NDC + dc
                return 0
            return lax.fori_loop(0, TCHUNK // LANES, mk_k, 0)
        lax.fori_loop(0, NCHUNK, mk_idx, 0)

        def gadd(i, _):
            gds = []
            for q in range(NBUF):
                j = i * NBUF + q

                @pl.when(i >= 1)
                def _(q=q):
                    pltpu.make_async_copy(
                        gbufs[q], acc.at[pl.ds(base, TCHUNK)], ssems[q]).wait()
                gds.append(pltpu.async_copy(
                    table_hbm.at[idx_v.at[j]], gbufs[q], gsems[q]))

            @pl.when(i == 0)
            def _():
                for m in range(S // SROWS):
                    pltpu.make_async_copy(
                        zbuf, acc.at[pl.ds(base, SROWS)], zsem).wait()

            for q in range(NBUF):
                j = i * NBUF + q
                gds[q].wait()
                pltpu.async_copy(gbufs[q], acc.at[sidx_v.at[j]], ssems[q],
                                 add=True)
            return 0
        lax.fori_loop(0, NCHUNK // NBUF, gadd, 0)

        for q in range(NBUF):
            pltpu.make_async_copy(
                gbufs[q], acc.at[pl.ds(base, TCHUNK)], ssems[q]).wait()

        NT = S // SROWS

        def din(tt):
            return pltpu.async_copy(
                acc.at[pl.ds(base + tt * SROWS, SROWS)], gbufs[tt % NBUF],
                gsems[tt % NBUF])

        def dout(tt):
            return pltpu.async_copy(
                gbufs[tt % NBUF],
                out_hbm.at[r, pl.ds(tt * SROWS, SROWS), pl.ds(dc * DCW, DCW)],
                osems[tt % NBUF])

        dins = {0: din(0), 1: din(1)}
        douts = {}
        for tt in range(NT):
            q = tt % NBUF
            if tt + 2 < NT:
                if tt + 2 >= NBUF:
                    douts[tt - 2].wait()
                dins[tt + 2] = din(tt + 2)
            dins[tt].wait()
            buf = gbufs[q]

            def srow(g, _):
                invv = inv_v[pl.ds(tt * SROWS + g * LANES, LANES)]
                for i2 in range(LANES):
                    splat = jnp.full((LANES,), invv[i2], jnp.float32)
                    for k in range(DCW // LANES):
                        sl = pl.ds(k * LANES, LANES)
                        buf[g * LANES + i2, sl] = buf[g * LANES + i2, sl] * splat
                return 0
            lax.fori_loop(0, SROWS // LANES, srow, 0)

            pltpu.async_copy(
                zbuf, acc.at[pl.ds(base + tt * SROWS, SROWS)], zsem)
            douts[tt] = dout(tt)
        for tt in range(NT - 4, NT):
            douts[tt].wait()
        return 0
    lax.fori_loop(0, NDC, dc_body, 0)

    for m in range(S // SROWS):
        pltpu.make_async_copy(zbuf, acc.at[pl.ds(base, SROWS)], zsem).wait()


def _pooled(tok2, seg2, table):
    mesh = plsc.VectorSubcoreMesh(
        core_axis_name="c", subcore_axis_name="s", num_cores=NC, num_subcores=NS)
    kern = pl.kernel(
        _body,
        out_type=(
            jax.ShapeDtypeStruct((R, S, D), jnp.float32),
            jax.ShapeDtypeStruct((R, 1, S), jnp.int32),
        ),
        mesh=mesh,
        compiler_params=pltpu.CompilerParams(needs_layout_passes=False),
        scratch_types=[
            pltpu.VMEM((NCHUNK, TCHUNK), jnp.int32),
            pltpu.VMEM((NCHUNK, TCHUNK), jnp.int32),
            pltpu.VMEM((NCHUNK, TCHUNK), jnp.int32),
            pltpu.VMEM((NCHUNK, TCHUNK), jnp.int32),
            pltpu.VMEM((TCHUNK, DCW), jnp.float32),
            pltpu.VMEM((TCHUNK, DCW), jnp.float32),
            pltpu.VMEM((TCHUNK, DCW), jnp.float32),
            pltpu.VMEM((TCHUNK, DCW), jnp.float32),
            pltpu.VMEM((SROWS, DCW), jnp.float32),
            pltpu.VMEM((L,), jnp.int32),
            pltpu.VMEM((EPAD + S,), jnp.int32),
            pltpu.VMEM((1, S), jnp.int32),
            pltpu.VMEM((S,), jnp.float32),
            pltpu.VMEM_SHARED((NS * S, DCW), jnp.float32),
            [pltpu.SemaphoreType.DMA] * 4,
            [pltpu.SemaphoreType.DMA] * 4,
            [pltpu.SemaphoreType.DMA] * 4,
            pltpu.SemaphoreType.DMA,
        ],
    )
    return kern(tok2, seg2, table)


def kernel(text_token_ids, text_seg_ids, amr_token_ids, amr_seg_ids, table):
    table8 = table.reshape(V * NDC, DCW)
    tok2 = jnp.concatenate(
        [text_token_ids.astype(jnp.int32), amr_token_ids.astype(jnp.int32)], axis=0
    ).reshape(R, NCHUNK, TCHUNK)
    seg2 = jnp.concatenate(
        [text_seg_ids.astype(jnp.int32), amr_seg_ids.astype(jnp.int32)], axis=0
    ).reshape(R, NCHUNK, TCHUNK)
    feats, cnts = _pooled(tok2, seg2, table8)
    pad = cnts[:, 0, :] == 0
    return feats[:B], pad[:B], feats[B:], pad[B:]


# speedup vs baseline: 1.5608x; 1.0016x over previous
import jax
import jax.numpy as jnp
from jax import lax
from jax.experimental import pallas as pl
from jax.experimental.pallas import tpu as pltpu
from jax.experimental.pallas import tpu_sc as plsc

B = 16
L = 2048
S = 512
V = 50265
D = 1024

NC = 2
NS = 16
LANES = 16

DCW = 128
NDC = D // DCW
TCHUNK = 32
NCHUNK = L // TCHUNK
NBUF = 8
SROWS = 32
R = 2 * B
NV = L // LANES
EPAD = LANES


def _body(tok_hbm, seg_hbm, table_hbm, out_hbm, cnt_hbm,
          seg_v, sidx_v, idx_v, gb0, gb1, gb2, gb3, gb4, gb5, gb6, gb7, zbuf,
          seg_f, ends_v, cntbuf, inv_v,
          acc, gsems, ssems, osems, zsem):
    c = lax.axis_index("c")
    s = lax.axis_index("s")
    r = c * NS + s
    base = s * S
    iota = lax.iota(jnp.int32, LANES)
    gbufs = (gb0, gb1, gb2, gb3, gb4, gb5, gb6, gb7)

    pltpu.sync_copy(tok_hbm.at[r], idx_v)
    pltpu.sync_copy(seg_hbm.at[r], seg_v)

    def mk_base_idx(j, _):
        def bk(k, _):
            sl = pl.ds(k * LANES, LANES)
            idx_v[j, sl] = idx_v[j, sl] * NDC
            return 0
        return lax.fori_loop(0, TCHUNK // LANES, bk, 0)
    lax.fori_loop(0, NCHUNK, mk_base_idx, 0)

    def mk_sidx(j, _):
        def mk_k(k, _):
            sl = pl.ds(k * LANES, LANES)
            sidx_v[j, sl] = seg_v[j, sl] + base
            return 0
        return lax.fori_loop(0, TCHUNK // LANES, mk_k, 0)
    lax.fori_loop(0, NCHUNK, mk_sidx, 0)

    def zero_ends(i, _):
        ends_v[pl.ds(i * LANES, LANES)] = jnp.zeros((LANES,), jnp.int32)
        return 0
    lax.fori_loop(0, (EPAD + S) // LANES, zero_ends, 0)

    def mk_flat(j, _):
        def fk(k, _):
            seg_f[pl.ds(j * TCHUNK + k * LANES, LANES)] = \
                seg_v[j, pl.ds(k * LANES, LANES)]
            return 0
        return lax.fori_loop(0, TCHUNK // LANES, fk, 0)
    lax.fori_loop(0, NCHUNK, mk_flat, 0)

    def bounds(t, _):
        cur = seg_f[pl.ds(t * LANES, LANES)]
        pos = t * LANES + iota
        pnx = jnp.minimum(pos + 1, L - 1)
        nxt = plsc.load_gather(seg_f, [pnx])
        mask = jnp.logical_or(cur != nxt, pos == L - 1)
        plsc.store_scatter(ends_v, [cur + EPAD], pos + 1, mask=mask)
        return 0
    lax.fori_loop(0, NV, bounds, 0)

    def fill(i, carry):
        sl = pl.ds(EPAD + i * LANES, LANES)
        filled = jnp.maximum(plsc.cummax(ends_v[sl]),
                             jnp.full((LANES,), carry, jnp.int32))
        ends_v[sl] = filled
        prev = plsc.load_gather(ends_v, [EPAD - 1 + i * LANES + iota])
        cnt_i = filled - prev
        cntbuf[0, pl.ds(i * LANES, LANES)] = cnt_i
        inv_v[pl.ds(i * LANES, LANES)] = 1.0 / jnp.maximum(
            cnt_i.astype(jnp.float32), 1.0)
        return jnp.max(filled)
    lax.fori_loop(0, S // LANES, fill, jnp.int32(0))

    pltpu.sync_copy(cntbuf, cnt_hbm.at[r])

    def zero_zb(i, _):
        def zk(k, _):
            zbuf[i, pl.ds(k * LANES, LANES)] = jnp.zeros((LANES,), jnp.float32)
            return 0
        return lax.fori_loop(0, DCW // LANES, zk, 0)
    lax.fori_loop(0, SROWS, zero_zb, 0)

    for m in range(S // SROWS):
        pltpu.async_copy(zbuf, acc.at[pl.ds(base + m * SROWS, SROWS)], zsem)

    def dc_body(dc, _):
        @pl.when(dc >= 1)
        def _():
            def mk_idx(j, _):
                def mk_k(k, _):
                    sl = pl.ds(k * LANES, LANES)
                    idx_v[j, sl] = idx_v[j, sl] + 1
                    return 0
                return lax.fori_loop(0, TCHUNK // LANES, mk_k, 0)
            lax.fori_loop(0, NCHUNK, mk_idx, 0)

        def gadd(i, _):
            gds = []
            for q in range(NBUF):
                j = i * NBUF + q

                @pl.when(i >= 1)
                def _(q=q):
                    pltpu.make_async_copy(
                        gbufs[q], acc.at[pl.ds(base, TCHUNK)], ssems[q]).wait()
                gds.append(pltpu.async_copy(
                    table_hbm.at[idx_v.at[j]], gbufs[q], gsems[q]))

            @pl.when(i == 0)
            def _():
                for m in range(S // SROWS):
                    pltpu.make_async_copy(
                        zbuf, acc.at[pl.ds(base, SROWS)], zsem).wait()

            for q in range(NBUF):
                j = i * NBUF + q
                gds[q].wait()
                pltpu.async_copy(gbufs[q], acc.at[sidx_v.at[j]], ssems[q],
                                 add=True)
            return 0
        lax.fori_loop(0, NCHUNK // NBUF, gadd, 0)

        for q in range(NBUF):
            pltpu.make_async_copy(
                gbufs[q], acc.at[pl.ds(base, TCHUNK)], ssems[q]).wait()

        NT = S // SROWS

        def din(tt):
            return pltpu.async_copy(
                acc.at[pl.ds(base + tt * SROWS, SROWS)], gbufs[tt % NBUF],
                gsems[tt % NBUF])

        def dout(tt):
            return pltpu.async_copy(
                gbufs[tt % NBUF],
                out_hbm.at[r, pl.ds(tt * SROWS, SROWS), pl.ds(dc * DCW, DCW)],
                osems[tt % NBUF])

        dins = {0: din(0), 1: din(1)}
        douts = {}
        for tt in range(NT):
            q = tt % NBUF
            if tt + 2 < NT:
                if tt + 2 - NBUF >= 0:
                    douts[tt + 2 - NBUF].wait()
                dins[tt + 2] = din(tt + 2)
            dins[tt].wait()
            buf = gbufs[q]

            def srow(g, _):
                invv = inv_v[pl.ds(tt * SROWS + g * LANES, LANES)]
                for i2 in range(LANES):
                    splat = jnp.full((LANES,), invv[i2], jnp.float32)
                    for k in range(DCW // LANES):
                        sl = pl.ds(k * LANES, LANES)
                        buf[g * LANES + i2, sl] = buf[g * LANES + i2, sl] * splat
                return 0
            lax.fori_loop(0, SROWS // LANES, srow, 0)

            pltpu.async_copy(
                zbuf, acc.at[pl.ds(base + tt * SROWS, SROWS)], zsem)
            douts[tt] = dout(tt)
        for tt in range(NT - NBUF, NT):
            douts[tt].wait()
        return 0
    lax.fori_loop(0, NDC, dc_body, 0)

    for m in range(S // SROWS):
        pltpu.make_async_copy(zbuf, acc.at[pl.ds(base, SROWS)], zsem).wait()


def _pooled(tok2, seg2, table):
    mesh = plsc.VectorSubcoreMesh(
        core_axis_name="c", subcore_axis_name="s", num_cores=NC, num_subcores=NS)
    kern = pl.kernel(
        _body,
        out_type=(
            jax.ShapeDtypeStruct((R, S, D), jnp.float32),
            jax.ShapeDtypeStruct((R, 1, S), jnp.int32),
        ),
        mesh=mesh,
        compiler_params=pltpu.CompilerParams(needs_layout_passes=False),
        scratch_types=[
            pltpu.VMEM((NCHUNK, TCHUNK), jnp.int32),
            pltpu.VMEM((NCHUNK, TCHUNK), jnp.int32),
            pltpu.VMEM((NCHUNK, TCHUNK), jnp.int32),
            pltpu.VMEM((TCHUNK, DCW), jnp.float32),
            pltpu.VMEM((TCHUNK, DCW), jnp.float32),
            pltpu.VMEM((TCHUNK, DCW), jnp.float32),
            pltpu.VMEM((TCHUNK, DCW), jnp.float32),
            pltpu.VMEM((TCHUNK, DCW), jnp.float32),
            pltpu.VMEM((TCHUNK, DCW), jnp.float32),
            pltpu.VMEM((TCHUNK, DCW), jnp.float32),
            pltpu.VMEM((TCHUNK, DCW), jnp.float32),
            pltpu.VMEM((SROWS, DCW), jnp.float32),
            pltpu.VMEM((L,), jnp.int32),
            pltpu.VMEM((EPAD + S,), jnp.int32),
            pltpu.VMEM((1, S), jnp.int32),
            pltpu.VMEM((S,), jnp.float32),
            pltpu.VMEM_SHARED((NS * S, DCW), jnp.float32),
            [pltpu.SemaphoreType.DMA] * 8,
            [pltpu.SemaphoreType.DMA] * 8,
            [pltpu.SemaphoreType.DMA] * 8,
            pltpu.SemaphoreType.DMA,
        ],
    )
    return kern(tok2, seg2, table)


def kernel(text_token_ids, text_seg_ids, amr_token_ids, amr_seg_ids, table):
    table8 = table.reshape(V * NDC, DCW)
    tok2 = jnp.concatenate(
        [text_token_ids.astype(jnp.int32), amr_token_ids.astype(jnp.int32)], axis=0
    ).reshape(R, NCHUNK, TCHUNK)
    seg2 = jnp.concatenate(
        [text_seg_ids.astype(jnp.int32), amr_seg_ids.astype(jnp.int32)], axis=0
    ).reshape(R, NCHUNK, TCHUNK)
    feats, cnts = _pooled(tok2, seg2, table8)
    pad = cnts[:, 0, :] == 0
    return feats[:B], pad[:B], feats[B:], pad[B:]
